# baseline (speedup 1.0000x reference)
import functools

import jax
import jax.numpy as jnp
from jax.experimental import pallas as pl
from jax.experimental.pallas import tpu as pltpu

_N = 10000
_F = 128
_FUSSY = 4
_BM = 400


def _proj_body(x_ref, w_ref, out_ref):
    out_ref[...] = jnp.dot(x_ref[...], w_ref[...],
                           preferred_element_type=jnp.float32
                           ).astype(jnp.bfloat16)


def _project(feat_bf, W_bf):
    return pl.pallas_call(
        _proj_body,
        out_shape=jax.ShapeDtypeStruct((_N, _F), jnp.bfloat16),
    )(feat_bf, W_bf)


_BH = _BM // 2


def _layer_body(mu_ref, nis_ref, adj0_ref, adj1_ref, y_ref, b_ref, wn_ref,
                out_ref, *, project_out):
    for s, a_ref in enumerate((adj0_ref, adj1_ref)):
        z = jnp.dot(a_ref[...].astype(jnp.bfloat16), y_ref[...],
                    preferred_element_type=jnp.float32)
        z = z + b_ref[...]
        acc = None
        for k in range(_FUSSY):
            d = z - mu_ref[0, k]
            t = jnp.exp(d * d * nis_ref[0, k])
            acc = t if acc is None else acc + t
        gated = z * (acc * (1.0 / _FUSSY))
        if project_out:
            res = jnp.dot(gated.astype(jnp.bfloat16), wn_ref[...],
                          preferred_element_type=jnp.float32
                          ).astype(jnp.bfloat16)
        else:
            res = gated
        out_ref[s * _BH:(s + 1) * _BH, :] = res


def _fused_layer(adj, y_bf, b, mu, sig, W_next_bf):
    project_out = W_next_bf is not None
    mu2d = mu.reshape(1, _FUSSY)
    neg_inv_sig2 = (-1.0 / (sig * sig)).reshape(1, _FUSSY)
    b2d = b.reshape(1, _F)
    if not project_out:
        W_next_bf = jnp.zeros((_F, _F), dtype=jnp.bfloat16)
    out_dtype = jnp.bfloat16 if project_out else jnp.float32
    return pl.pallas_call(
        functools.partial(_layer_body, project_out=project_out),
        grid=(_N // _BM,),
        in_specs=[
            pl.BlockSpec(memory_space=pltpu.SMEM),
            pl.BlockSpec(memory_space=pltpu.SMEM),
            pl.BlockSpec((_BH, _N), lambda i: (2 * i, 0)),
            pl.BlockSpec((_BH, _N), lambda i: (2 * i + 1, 0)),
            pl.BlockSpec((_N, _F), lambda i: (0, 0)),
            pl.BlockSpec((1, _F), lambda i: (0, 0)),
            pl.BlockSpec((_F, _F), lambda i: (0, 0)),
        ],
        out_specs=pl.BlockSpec((_BM, _F), lambda i: (i, 0)),
        out_shape=jax.ShapeDtypeStruct((_N, _F), out_dtype),
        compiler_params=pltpu.CompilerParams(
            vmem_limit_bytes=100 * 1024 * 1024,
        ),
    )(mu2d, neg_inv_sig2, adj, adj, y_bf, b2d, W_next_bf)


def kernel(x, adj, W1, b1, mu1, sig1, W2, b2, mu2, sig2):
    y1 = _project(x.astype(jnp.bfloat16), W1.astype(jnp.bfloat16))
    y2 = _fused_layer(adj, y1, b1, mu1, sig1, W2.astype(jnp.bfloat16))
    return _fused_layer(adj, y2, b2, mu2, sig2, None)


# speedup vs baseline: 1.0695x; 1.0695x over previous
import jax
import jax.numpy as jnp
from jax.experimental import pallas as pl
from jax.experimental.pallas import tpu as pltpu

_N = 10000
_F = 128
_FUSSY = 4
_BM = 400


def _body(mu_ref, nis_ref, x_ref, adj_ref, w1_ref, w2_ref, b_ref, out_ref,
          y_ref, y2_ref):
    l = pl.program_id(0)
    i = pl.program_id(1)

    @pl.when(jnp.logical_and(l == 0, i == 0))
    def _init_y1():
        y_ref[...] = jnp.dot(x_ref[...].astype(jnp.bfloat16),
                             w1_ref[...].astype(jnp.bfloat16),
                             preferred_element_type=jnp.float32
                             ).astype(jnp.bfloat16)

    @pl.when(jnp.logical_and(l == 1, i == 0))
    def _swap_to_y2():
        y_ref[...] = y2_ref[...]

    z = jnp.dot(adj_ref[...].astype(jnp.bfloat16), y_ref[...],
                preferred_element_type=jnp.float32)
    z = z + b_ref[pl.ds(l, 1), :]
    acc = None
    for k in range(_FUSSY):
        d = z - mu_ref[l, k]
        t = jnp.exp(d * d * nis_ref[l, k])
        acc = t if acc is None else acc + t
    gated = z * (acc * (1.0 / _FUSSY))

    @pl.when(l == 0)
    def _store_layer1():
        y2_ref[pl.ds(i * _BM, _BM), :] = jnp.dot(
            gated.astype(jnp.bfloat16), w2_ref[...].astype(jnp.bfloat16),
            preferred_element_type=jnp.float32).astype(jnp.bfloat16)

    @pl.when(l == 1)
    def _store_layer2():
        out_ref[...] = gated


def kernel(x, adj, W1, b1, mu1, sig1, W2, b2, mu2, sig2):
    mu = jnp.stack([mu1, mu2])
    nis = -1.0 / jnp.stack([sig1 * sig1, sig2 * sig2])
    b = jnp.stack([b1, b2])
    return pl.pallas_call(
        _body,
        grid=(2, _N // _BM),
        in_specs=[
            pl.BlockSpec(memory_space=pltpu.SMEM),
            pl.BlockSpec(memory_space=pltpu.SMEM),
            pl.BlockSpec((_N, _F), lambda l, i: (0, 0)),
            pl.BlockSpec((_BM, _N), lambda l, i: (i, 0)),
            pl.BlockSpec((_F, _F), lambda l, i: (0, 0)),
            pl.BlockSpec((_F, _F), lambda l, i: (0, 0)),
            pl.BlockSpec((2, _F), lambda l, i: (0, 0)),
        ],
        out_specs=pl.BlockSpec((_BM, _F), lambda l, i: (i * l, 0)),
        out_shape=jax.ShapeDtypeStruct((_N, _F), jnp.float32),
        scratch_shapes=[
            pltpu.VMEM((_N, _F), jnp.bfloat16),
            pltpu.VMEM((_N, _F), jnp.bfloat16),
        ],
        compiler_params=pltpu.CompilerParams(
            vmem_limit_bytes=100 * 1024 * 1024,
        ),
    )(mu, nis, x, adj, W1, W2, b)
